# Initial kernel scaffold; baseline (speedup 1.0000x reference)
import jax
import jax.numpy as jnp
from jax.experimental import pallas as pl


def kernel(support_xyz, support_features, W, b):
    raise NotImplementedError("write your pallas kernel here")


# speedup vs baseline: 6.6613x; 6.6613x over previous
import functools

import jax
import jax.numpy as jnp
from jax.experimental import pallas as pl
from jax.experimental.pallas import tpu as pltpu

K_NN = 32
BQ = 256


def _uz_kernel(xyzT_ref, feat_ref, w3_ref, wf_ref, u_ref, z_ref):
    xyzT = xyzT_ref[0]
    z = jax.lax.dot_general(
        xyzT, w3_ref[...], (((0,), (0,)), ((), ())),
        preferred_element_type=jnp.float32)
    u = z + jnp.dot(feat_ref[0], wf_ref[...],
                    preferred_element_type=jnp.float32)
    u_ref[0] = u
    z_ref[0] = z


def _knn_kernel(qT_ref, sT_ref, u_ref, z_ref, b_ref, o_ref):
    qT = qT_ref[0]
    sT = sT_ref[0]
    cross = jax.lax.dot_general(
        qT, sT, (((0,), (0,)), ((), ())),
        preferred_element_type=jnp.float32)
    qn = jnp.sum(qT * qT, axis=0)[:, None]
    sn = jnp.sum(sT * sT, axis=0)[None, :]
    d0 = qn + sn - 2.0 * cross
    n = d0.shape[1]
    iota = jax.lax.broadcasted_iota(jnp.int32, d0.shape, 1)
    u16 = u_ref[0].astype(jnp.bfloat16)

    def body(_, carry):
        d, acc = carry
        m = jnp.min(d, axis=1, keepdims=True)
        cand = jnp.where(d == m, iota, n)
        am = jnp.min(cand, axis=1, keepdims=True)
        onehot = iota == am
        sel = jax.lax.dot_general(
            jnp.where(onehot, 1.0, 0.0).astype(jnp.bfloat16), u16,
            (((1,), (0,)), ((), ())),
            preferred_element_type=jnp.float32)
        acc = jnp.maximum(acc, sel)
        d = jnp.where(onehot, jnp.inf, d)
        return d, acc

    _, acc = jax.lax.fori_loop(
        0, K_NN, body,
        (d0, jnp.full((d0.shape[0], u16.shape[1]), -jnp.inf, jnp.float32)))
    o_ref[0] = jnp.maximum(acc - z_ref[0] + b_ref[...], 0.0)


def kernel(support_xyz, support_features, W, b):
    B, N, _ = support_xyz.shape
    C = support_features.shape[-1]
    C_OUT = W.shape[-1]
    xyzT = jnp.transpose(support_xyz, (0, 2, 1))
    w3 = W[:3]
    wf = W[3:]

    u, z = pl.pallas_call(
        _uz_kernel,
        grid=(B,),
        in_specs=[
            pl.BlockSpec((1, 3, N), lambda i: (i, 0, 0)),
            pl.BlockSpec((1, N, C), lambda i: (i, 0, 0)),
            pl.BlockSpec((3, C_OUT), lambda i: (0, 0)),
            pl.BlockSpec((C, C_OUT), lambda i: (0, 0)),
        ],
        out_specs=[
            pl.BlockSpec((1, N, C_OUT), lambda i: (i, 0, 0)),
            pl.BlockSpec((1, N, C_OUT), lambda i: (i, 0, 0)),
        ],
        out_shape=[
            jax.ShapeDtypeStruct((B, N, C_OUT), jnp.float32),
            jax.ShapeDtypeStruct((B, N, C_OUT), jnp.float32),
        ],
    )(xyzT, support_features, w3, wf)

    b2 = b[None, :]
    new_features = pl.pallas_call(
        _knn_kernel,
        grid=(B, N // BQ),
        in_specs=[
            pl.BlockSpec((1, 3, BQ), lambda i, j: (i, 0, j)),
            pl.BlockSpec((1, 3, N), lambda i, j: (i, 0, 0)),
            pl.BlockSpec((1, N, C_OUT), lambda i, j: (i, 0, 0)),
            pl.BlockSpec((1, BQ, C_OUT), lambda i, j: (i, j, 0)),
            pl.BlockSpec((1, C_OUT), lambda i, j: (0, 0)),
        ],
        out_specs=pl.BlockSpec((1, BQ, C_OUT), lambda i, j: (i, j, 0)),
        out_shape=jax.ShapeDtypeStruct((B, N, C_OUT), jnp.float32),
    )(xyzT, xyzT, u, z, b2)

    return (support_xyz, new_features)


# speedup vs baseline: 17.9342x; 2.6923x over previous
import functools

import jax
import jax.numpy as jnp
from jax import lax
from jax.experimental import pallas as pl
from jax.experimental.pallas import tpu as pltpu
from jax.experimental.pallas import tpu_sc as plsc

K_NN = 32
BQ = 256
NCHUNK = 64
M_PER_CHUNK = 6
IMAX = 2147483647


def _uz_kernel(xyzT_ref, feat_ref, w3_ref, wf_ref, b_ref, u_ref, z_ref):
    xyzT = xyzT_ref[0]
    z = lax.dot_general(xyzT, w3_ref[...], (((0,), (0,)), ((), ())),
                        preferred_element_type=jnp.float32)
    u = z + jnp.dot(feat_ref[0], wf_ref[...],
                    preferred_element_type=jnp.float32)
    u_ref[0] = u
    z_ref[0] = z - b_ref[...]


def _sortable(x):
    k = lax.bitcast_convert_type(x, jnp.int32)
    return k ^ jnp.where(k < 0, jnp.int32(0x7FFFFFFF), jnp.int32(0))


def _tree_min(x, width):
    while x.shape[0] > width:
        h = x.shape[0] // 2
        x = jnp.minimum(x[:h], x[h:])
    return x


def _tile_rows(x, rows):
    while x.shape[0] < rows:
        x = jnp.concatenate([x, x], axis=0)
    return x


def _sel_kernel(qT_ref, sT_ref, o_ref, sel_ref):
    n = sT_ref.shape[2]
    bq = qT_ref.shape[2]
    sT = sT_ref[0]
    qT = qT_ref[0]
    snc = jnp.transpose(jnp.sum(sT * sT, axis=0, keepdims=True))
    cross = lax.dot_general(sT, qT, (((0,), (0,)), ((), ())),
                            preferred_element_type=jnp.float32)
    dT = snc - 2.0 * cross
    key = _sortable(dT)
    ri = lax.broadcasted_iota(jnp.int32, (n, bq), 0)
    sub = lax.shift_right_logical(ri, 6)
    keyp = (key & jnp.int32(~(NCHUNK - 1))) | sub

    a = keyp
    cands_ek, cands_ix = [], []
    chunk_id64 = lax.broadcasted_iota(jnp.int32, (NCHUNK, bq), 0)
    for _ in range(M_PER_CHUNK):
        cm = _tree_min(a, NCHUNK)
        cmt = _tile_rows(cm, n)
        hit = a == cmt
        ek = _tree_min(jnp.where(hit, key, IMAX), NCHUNK)
        cands_ek.append(ek)
        cands_ix.append(((cm & jnp.int32(NCHUNK - 1)) << 6) | chunk_id64)
        a = jnp.where(hit, IMAX, a)
    cek = jnp.concatenate(cands_ek, axis=0)
    cix = jnp.concatenate(cands_ix, axis=0)

    def p2_body(s, carry):
        cek_w, _ = carry
        m2 = jnp.min(cek_w, axis=0)
        selm = cek_w == m2[None, :]
        idxs = jnp.min(jnp.where(selm, cix, IMAX), axis=0)
        sel_ref[pl.ds(s, 1), :] = idxs[None, :]
        cek_w = jnp.where(selm & (cix == idxs[None, :]), IMAX, cek_w)
        return cek_w, m2

    _, t = lax.fori_loop(0, K_NN, p2_body,
                         (cek, jnp.zeros((bq,), jnp.int32)))
    cnt = jnp.sum(jnp.where(key <= t[None, :], 1, 0), axis=0)
    ok = jnp.all(cnt == 32)

    @pl.when(jnp.logical_not(ok))
    def exact_fallback():
        def fb_body(s, fkw):
            m = jnp.min(fkw, axis=0)
            selm = fkw == m[None, :]
            idxs = jnp.min(jnp.where(selm, ri, IMAX), axis=0)
            sel_ref[pl.ds(s, 1), :] = idxs[None, :]
            return jnp.where(selm & (ri == idxs[None, :]), IMAX, fkw)

        lax.fori_loop(0, K_NN, fb_body, key)

    o_ref[0] = sel_ref[...] + pl.program_id(0) * n


def _make_sc_gather_max(nq_total, c_out, k_nn, qpw, gq):
    mesh = plsc.VectorSubcoreMesh(core_axis_name="c", subcore_axis_name="s")
    n_groups = qpw // gq
    gi = gq * k_nn

    @functools.partial(
        pl.kernel,
        out_type=jax.ShapeDtypeStruct((nq_total, c_out), jnp.float32),
        mesh=mesh,
        scratch_types=[
            pltpu.VMEM((gi,), jnp.int32),
            pltpu.VMEM((gi, c_out), jnp.float32),
            pltpu.VMEM((gq, c_out), jnp.float32),
            pltpu.VMEM((gq, c_out), jnp.float32),
            pltpu.SemaphoreType.DMA,
        ],
    )
    def sc_kernel(u_hbm, idx_hbm, zb_hbm, out_hbm, idx_v, rows_v, zb_v,
                  out_v, sem):
        wid = lax.axis_index("s") * 2 + lax.axis_index("c")
        qbase0 = wid * qpw

        @pl.loop(0, n_groups)
        def _(g):
            qb = qbase0 + g * gq
            pltpu.sync_copy(idx_hbm.at[pl.ds(qb * k_nn, gi)], idx_v)
            pltpu.async_copy(u_hbm.at[idx_v], rows_v, sem).wait()
            pltpu.sync_copy(zb_hbm.at[pl.ds(qb, gq)], zb_v)
            for q in range(gq):
                for c in range(c_out // 16):
                    cs = pl.ds(c * 16, 16)

                    def mbody(r, acc):
                        return jnp.maximum(acc, rows_v[q * k_nn + r, cs])

                    acc = lax.fori_loop(1, k_nn, mbody,
                                        rows_v[q * k_nn, cs])
                    out_v[q, cs] = jnp.maximum(acc - zb_v[q, cs], 0.0)
            pltpu.sync_copy(out_v, out_hbm.at[pl.ds(qb, gq)])

    return sc_kernel


def kernel(support_xyz, support_features, W, b):
    B, N, _ = support_xyz.shape
    C = support_features.shape[-1]
    C_OUT = W.shape[-1]
    xyzT = jnp.transpose(support_xyz, (0, 2, 1))

    u, zb = pl.pallas_call(
        _uz_kernel,
        grid=(B,),
        in_specs=[
            pl.BlockSpec((1, 3, N), lambda i: (i, 0, 0)),
            pl.BlockSpec((1, N, C), lambda i: (i, 0, 0)),
            pl.BlockSpec((3, C_OUT), lambda i: (0, 0)),
            pl.BlockSpec((C, C_OUT), lambda i: (0, 0)),
            pl.BlockSpec((1, C_OUT), lambda i: (0, 0)),
        ],
        out_specs=[
            pl.BlockSpec((1, N, C_OUT), lambda i: (i, 0, 0)),
            pl.BlockSpec((1, N, C_OUT), lambda i: (i, 0, 0)),
        ],
        out_shape=[
            jax.ShapeDtypeStruct((B, N, C_OUT), jnp.float32),
            jax.ShapeDtypeStruct((B, N, C_OUT), jnp.float32),
        ],
    )(xyzT, support_features, W[:3], W[3:], b[None, :])

    idx_t = pl.pallas_call(
        _sel_kernel,
        grid=(B, N // BQ),
        in_specs=[
            pl.BlockSpec((1, 3, BQ), lambda i, j: (i, 0, j)),
            pl.BlockSpec((1, 3, N), lambda i, j: (i, 0, 0)),
        ],
        out_specs=pl.BlockSpec((1, K_NN, BQ), lambda i, j: (i, 0, j)),
        out_shape=jax.ShapeDtypeStruct((B, K_NN, N), jnp.int32),
        scratch_shapes=[pltpu.VMEM((K_NN, BQ), jnp.int32)],
    )(xyzT, xyzT)
    idx = jnp.transpose(idx_t, (0, 2, 1))

    nq_total = B * N
    u_flat = u.reshape(nq_total, C_OUT)
    zb_flat = zb.reshape(nq_total, C_OUT)
    idx_flat = idx.reshape(nq_total * K_NN)
    sc = _make_sc_gather_max(nq_total, C_OUT, K_NN, nq_total // 32, 4)
    out_flat = sc(u_flat, idx_flat, zb_flat)
    return (support_xyz, out_flat.reshape(B, N, C_OUT))


# speedup vs baseline: 25.9069x; 1.4446x over previous
import functools

import jax
import jax.numpy as jnp
from jax import lax
from jax.experimental import pallas as pl
from jax.experimental.pallas import tpu as pltpu
from jax.experimental.pallas import tpu_sc as plsc

K_NN = 32
BQ = 256
NCHUNK = 64
M_PER_CHUNK = 6
IMAX = 2147483647


def _uz_kernel(xyzT_ref, feat_ref, w3_ref, wf_ref, b_ref, u_ref, z_ref):
    xyzT = xyzT_ref[0]
    z = lax.dot_general(xyzT, w3_ref[...], (((0,), (0,)), ((), ())),
                        preferred_element_type=jnp.float32)
    u = z + jnp.dot(feat_ref[0], wf_ref[...],
                    preferred_element_type=jnp.float32)
    u_ref[0] = u
    z_ref[0] = z - b_ref[...]


def _sortable(x):
    k = lax.bitcast_convert_type(x, jnp.int32)
    return k ^ jnp.where(k < 0, jnp.int32(0x7FFFFFFF), jnp.int32(0))


def _tree_min(x, width):
    while x.shape[0] > width:
        h = x.shape[0] // 2
        x = jnp.minimum(x[:h], x[h:])
    return x


def _tile_rows(x, rows):
    while x.shape[0] < rows:
        x = jnp.concatenate([x, x], axis=0)
    return x


def _sel_kernel(off_ref, qT_ref, sT_ref, o_ref, sel_ref):
    n = sT_ref.shape[1]
    bq = qT_ref.shape[1]
    sT = sT_ref[...]
    qT = qT_ref[...]
    snc = jnp.transpose(jnp.sum(sT * sT, axis=0, keepdims=True))
    cross = lax.dot_general(sT, qT, (((0,), (0,)), ((), ())),
                            preferred_element_type=jnp.float32)
    dT = snc - 2.0 * cross
    key = _sortable(dT)
    ri = lax.broadcasted_iota(jnp.int32, (n, bq), 0)
    sub = lax.shift_right_logical(ri, 6)
    keyp = (key & jnp.int32(~(NCHUNK - 1))) | sub

    a = keyp
    cands_ek, cands_ix = [], []
    chunk_id64 = lax.broadcasted_iota(jnp.int32, (NCHUNK, bq), 0)
    for _ in range(M_PER_CHUNK):
        cm = _tree_min(a, NCHUNK)
        cmt = _tile_rows(cm, n)
        hit = a == cmt
        ek = _tree_min(jnp.where(hit, key, IMAX), NCHUNK)
        cands_ek.append(ek)
        cands_ix.append(((cm & jnp.int32(NCHUNK - 1)) << 6) | chunk_id64)
        a = jnp.where(hit, IMAX, a)
    cek = jnp.concatenate(cands_ek, axis=0)
    cix = jnp.concatenate(cands_ix, axis=0)

    def p2_body(s, carry):
        cek_w, _ = carry
        m2 = jnp.min(cek_w, axis=0)
        selm = cek_w == m2[None, :]
        idxs = jnp.min(jnp.where(selm, cix, IMAX), axis=0)
        sel_ref[pl.ds(s, 1), :] = idxs[None, :]
        cek_w = jnp.where(selm & (cix == idxs[None, :]), IMAX, cek_w)
        return cek_w, m2

    _, t = lax.fori_loop(0, K_NN, p2_body,
                         (cek, jnp.zeros((bq,), jnp.int32)))
    cnt = jnp.sum(jnp.where(key <= t[None, :], 1, 0), axis=0)
    ok = jnp.all(cnt == 32)

    @pl.when(jnp.logical_not(ok))
    def exact_fallback():
        def fb_body(s, fkw):
            m = jnp.min(fkw, axis=0)
            selm = fkw == m[None, :]
            idxs = jnp.min(jnp.where(selm, ri, IMAX), axis=0)
            sel_ref[pl.ds(s, 1), :] = idxs[None, :]
            return jnp.where(selm & (ri == idxs[None, :]), IMAX, fkw)

        lax.fori_loop(0, K_NN, fb_body, key)

    o_ref[...] = sel_ref[...] + off_ref[0]


def _make_sc_gather_max(nq, c_out, k_nn, n_workers, gq):
    qpw = nq // n_workers
    n_groups = qpw // gq
    gi = gq * k_nn
    nch = c_out // 16
    mesh = plsc.VectorSubcoreMesh(core_axis_name="c", subcore_axis_name="s")

    @functools.partial(
        pl.kernel,
        out_type=jax.ShapeDtypeStruct((nq, c_out), jnp.float32),
        mesh=mesh,
        scratch_types=[
            pltpu.VMEM((n_groups, gi), jnp.int32),
            pltpu.VMEM((gi, c_out), jnp.float32),
            pltpu.VMEM((gi, c_out), jnp.float32),
            pltpu.VMEM((gq, c_out), jnp.float32),
            pltpu.VMEM((gq, c_out), jnp.float32),
            pltpu.SemaphoreType.DMA,
            pltpu.SemaphoreType.DMA,
        ],
    )
    def sc_kernel(u_hbm, idx_hbm, zb_hbm, out_hbm, idx_v, rows0, rows1,
                  zb_v, out_v, sem0, sem1):
        wid = lax.axis_index("s") * 2 + lax.axis_index("c")
        qbase0 = wid * qpw
        pltpu.sync_copy(idx_hbm.at[wid], idx_v)
        pltpu.async_copy(u_hbm.at[idx_v.at[0]], rows0, sem0)

        def compute(rows_v, g):
            qb = qbase0 + g * gq
            pltpu.sync_copy(zb_hbm.at[pl.ds(qb, gq)], zb_v)
            for q in range(gq):
                base = q * k_nn

                def mbody(r, accs):
                    return tuple(
                        jnp.maximum(accs[c],
                                    rows_v[base + r, pl.ds(c * 16, 16)])
                        for c in range(nch))

                accs = tuple(rows_v[base, pl.ds(c * 16, 16)]
                             for c in range(nch))
                accs = lax.fori_loop(1, k_nn, mbody, accs)
                for c in range(nch):
                    cs = pl.ds(c * 16, 16)
                    out_v[q, cs] = jnp.maximum(accs[c] - zb_v[q, cs], 0.0)
            pltpu.sync_copy(out_v, out_hbm.at[pl.ds(qb, gq)])

        @pl.loop(0, n_groups, step=2)
        def _(g):
            pltpu.async_copy(u_hbm.at[idx_v.at[g + 1]], rows1, sem1)
            pltpu.make_async_copy(u_hbm.at[idx_v.at[g]], rows0, sem0).wait()
            compute(rows0, g)

            @pl.when(g + 2 < n_groups)
            def _():
                pltpu.async_copy(u_hbm.at[idx_v.at[g + 2]], rows0, sem0)

            pltpu.make_async_copy(u_hbm.at[idx_v.at[g + 1]], rows1,
                                  sem1).wait()
            compute(rows1, g + 1)

    return sc_kernel


def kernel(support_xyz, support_features, W, b):
    B, N, _ = support_xyz.shape
    C = support_features.shape[-1]
    C_OUT = W.shape[-1]
    n_workers, gq = 32, 4
    xyzT = jnp.transpose(support_xyz, (0, 2, 1))

    u, zb = pl.pallas_call(
        _uz_kernel,
        grid=(B,),
        in_specs=[
            pl.BlockSpec((1, 3, N), lambda i: (i, 0, 0)),
            pl.BlockSpec((1, N, C), lambda i: (i, 0, 0)),
            pl.BlockSpec((3, C_OUT), lambda i: (0, 0)),
            pl.BlockSpec((C, C_OUT), lambda i: (0, 0)),
            pl.BlockSpec((1, C_OUT), lambda i: (0, 0)),
        ],
        out_specs=[
            pl.BlockSpec((1, N, C_OUT), lambda i: (i, 0, 0)),
            pl.BlockSpec((1, N, C_OUT), lambda i: (i, 0, 0)),
        ],
        out_shape=[
            jax.ShapeDtypeStruct((B, N, C_OUT), jnp.float32),
            jax.ShapeDtypeStruct((B, N, C_OUT), jnp.float32),
        ],
    )(xyzT, support_features, W[:3], W[3:], b[None, :])

    u_flat = u.reshape(B * N, C_OUT)
    sel_call = functools.partial(
        pl.pallas_call,
        _sel_kernel,
        grid=(N // BQ,),
        in_specs=[
            pl.BlockSpec(memory_space=pltpu.SMEM),
            pl.BlockSpec((3, BQ), lambda j: (0, j)),
            pl.BlockSpec((3, N), lambda j: (0, 0)),
        ],
        out_specs=pl.BlockSpec((K_NN, BQ), lambda j: (0, j)),
        out_shape=jax.ShapeDtypeStruct((K_NN, N), jnp.int32),
        scratch_shapes=[pltpu.VMEM((K_NN, BQ), jnp.int32)],
    )
    sc = _make_sc_gather_max(N, C_OUT, K_NN, n_workers, gq)

    outs = []
    for bi in range(B):
        off = jnp.full((1,), bi * N, jnp.int32)
        idx_t = sel_call()(off, xyzT[bi], xyzT[bi])
        idx_w = jnp.transpose(idx_t, (1, 0)).reshape(
            n_workers, N // n_workers // gq, gq * K_NN)
        outs.append(sc(u_flat, idx_w, zb[bi]))
    out = jnp.stack(outs)
    return (support_xyz, out)
